# Initial kernel scaffold; baseline (speedup 1.0000x reference)
import jax
import jax.numpy as jnp
from jax.experimental import pallas as pl


def kernel(input, weight):
    raise NotImplementedError("write your pallas kernel here")


# speedup vs baseline: 4.9473x; 4.9473x over previous
import functools

import jax
import jax.numpy as jnp
from jax import lax
from jax.experimental import pallas as pl
from jax.experimental.pallas import tpu as pltpu
from jax.experimental.pallas import tpu_sc as plsc

_info = plsc.get_sparse_core_info()
_NC, _NS = _info.num_cores, _info.num_subcores
_NW = _NC * _NS


def _make_gather(B: int, D: int, chunk: int):
    assert B % (_NW * chunk) == 0
    b_per_w = B // _NW
    n_chunks = b_per_w // chunk
    mesh = plsc.VectorSubcoreMesh(core_axis_name="c", subcore_axis_name="s")

    @functools.partial(
        pl.kernel,
        mesh=mesh,
        out_type=jax.ShapeDtypeStruct((B, D), jnp.float32),
        compiler_params=pltpu.CompilerParams(use_tc_tiling_on_sc=False),
        scratch_types=[
            pltpu.VMEM((chunk,), jnp.int32),
            pltpu.VMEM((chunk, D), jnp.float32),
            pltpu.SemaphoreType.DMA,
        ],
    )
    def gather(idx_hbm, table_hbm, out_hbm, idx_v, rows_v, sem):
        wid = lax.axis_index("s") * _NC + lax.axis_index("c")
        base = wid * b_per_w

        def body(j, carry):
            off = base + j * chunk
            pltpu.sync_copy(idx_hbm.at[pl.ds(off, chunk)], idx_v)
            pltpu.async_copy(table_hbm.at[idx_v], rows_v, sem).wait()
            pltpu.sync_copy(rows_v, out_hbm.at[pl.ds(off, chunk)])
            return carry

        lax.fori_loop(0, n_chunks, body, 0)

    return gather


def kernel(input, weight):
    b, s = input.shape
    vocab, d = weight.shape
    flat_idx = input.reshape(b * s)
    out = _make_gather(b * s, d, 2048)(flat_idx, weight)
    return out.reshape(b, s, d)


# speedup vs baseline: 5.0506x; 1.0209x over previous
import functools

import jax
import jax.numpy as jnp
from jax import lax
from jax.experimental import pallas as pl
from jax.experimental.pallas import tpu as pltpu
from jax.experimental.pallas import tpu_sc as plsc

_info = plsc.get_sparse_core_info()
_NC, _NS = _info.num_cores, _info.num_subcores
_NW = _NC * _NS


def _make_gather(B: int, D: int, chunk: int):
    assert B % (_NW * chunk) == 0
    b_per_w = B // _NW
    n_chunks = b_per_w // chunk
    assert n_chunks >= 4 and (n_chunks - 2) % 2 == 0
    mesh = plsc.VectorSubcoreMesh(core_axis_name="c", subcore_axis_name="s")

    @functools.partial(
        pl.kernel,
        mesh=mesh,
        out_type=jax.ShapeDtypeStruct((B, D), jnp.float32),
        compiler_params=pltpu.CompilerParams(use_tc_tiling_on_sc=False),
        scratch_types=[
            pltpu.VMEM((chunk,), jnp.int32),
            pltpu.VMEM((chunk,), jnp.int32),
            pltpu.VMEM((chunk, D), jnp.float32),
            pltpu.VMEM((chunk, D), jnp.float32),
            pltpu.SemaphoreType.DMA,
            pltpu.SemaphoreType.DMA,
            pltpu.SemaphoreType.DMA,
            pltpu.SemaphoreType.DMA,
            pltpu.SemaphoreType.DMA,
            pltpu.SemaphoreType.DMA,
        ],
    )
    def gather(idx_hbm, table_hbm, out_hbm, idx0, idx1, rows0, rows1,
               gsem0, gsem1, osem0, osem1, isem0, isem1):
        wid = lax.axis_index("s") * _NC + lax.axis_index("c")
        base = wid * b_per_w
        idx_v = (idx0, idx1)
        rows_v = (rows0, rows1)
        gsem = (gsem0, gsem1)
        osem = (osem0, osem1)
        isem = (isem0, isem1)

        pltpu.sync_copy(idx_hbm.at[pl.ds(base, chunk)], idx0)
        pltpu.sync_copy(idx_hbm.at[pl.ds(base + chunk, chunk)], idx1)
        pltpu.async_copy(table_hbm.at[idx0], rows0, gsem0)
        pltpu.async_copy(table_hbm.at[idx1], rows1, gsem1)

        def half(c, s):
            off = base + c * chunk
            off2 = off + 2 * chunk
            pltpu.make_async_copy(table_hbm.at[idx_v[s]], rows_v[s], gsem[s]).wait()
            pltpu.async_copy(rows_v[s], out_hbm.at[pl.ds(off, chunk)], osem[s])
            pltpu.async_copy(idx_hbm.at[pl.ds(off2, chunk)], idx_v[s], isem[s])
            pltpu.make_async_copy(rows_v[s], out_hbm.at[pl.ds(off, chunk)], osem[s]).wait()
            pltpu.make_async_copy(idx_hbm.at[pl.ds(off2, chunk)], idx_v[s], isem[s]).wait()
            pltpu.async_copy(table_hbm.at[idx_v[s]], rows_v[s], gsem[s])

        def body(i, carry):
            half(2 * i, 0)
            half(2 * i + 1, 1)
            return carry

        lax.fori_loop(0, (n_chunks - 2) // 2, body, 0)

        for c in (n_chunks - 2, n_chunks - 1):
            s = c % 2
            off = base + c * chunk
            pltpu.make_async_copy(table_hbm.at[idx_v[s]], rows_v[s], gsem[s]).wait()
            pltpu.async_copy(rows_v[s], out_hbm.at[pl.ds(off, chunk)], osem[s])
        for c in (n_chunks - 2, n_chunks - 1):
            s = c % 2
            off = base + c * chunk
            pltpu.make_async_copy(rows_v[s], out_hbm.at[pl.ds(off, chunk)], osem[s]).wait()

    return gather


def kernel(input, weight):
    b, s = input.shape
    vocab, d = weight.shape
    flat_idx = input.reshape(b * s)
    out = _make_gather(b * s, d, 1600)(flat_idx, weight)
    return out.reshape(b, s, d)
